# Initial kernel scaffold; baseline (speedup 1.0000x reference)
import jax
import jax.numpy as jnp
from jax.experimental import pallas as pl


def kernel(x1, x2, w_t_0, b_0, bn_mean_0, bn_var_0, w_t_1, b_1, bn_mean_1, bn_var_1, w_t_2, b_2, bn_mean_2, bn_var_2, w_t_3, b_3, bn_mean_3, bn_var_3, w_t_4, b_4, bn_mean_4, bn_var_4, w_t_5, b_5, bn_mean_5, bn_var_5, w_t_6, b_6, bn_mean_6, bn_var_6, w_t_7, b_7, bn_mean_7, bn_var_7, w_t_8, b_8, bn_mean_8, bn_var_8, w_t_9, b_9, bn_mean_9, bn_var_9, w11_t, b11, wlda_t, blda, wplda_t, bplda, p_sqrt, q):
    raise NotImplementedError("write your pallas kernel here")


# speedup vs baseline: 1.0141x; 1.0141x over previous
import functools

import jax
import jax.numpy as jnp
from jax import lax
from jax.experimental import pallas as pl
from jax.experimental.pallas import tpu as pltpu

_BN_EPS = 1e-5

_CFG = [
    (30, 512, 5, 1), (512, 512, 1, 1), (512, 512, 3, 2), (512, 512, 1, 1),
    (512, 512, 3, 3), (512, 512, 1, 1), (512, 512, 3, 4), (512, 512, 1, 1),
    (512, 512, 1, 1), (512, 1500, 1, 1),
]
_N_LAYERS = len(_CFG)


def _trunk_body(x_ref, *args):
    ws = args[:_N_LAYERS]
    ps = args[_N_LAYERS:2 * _N_LAYERS]
    mean_ref, std_ref = args[2 * _N_LAYERS], args[2 * _N_LAYERS + 1]

    x = x_ref[0].astype(jnp.bfloat16)
    y = None
    for li, (d_in, d_out, ctx, dil) in enumerate(_CFG):
        t_in = x.shape[0]
        t_out = t_in - dil * (ctx - 1)
        if ctx == 1:
            xs = x
        else:
            xs = jnp.concatenate(
                [x[i * dil:i * dil + t_out, :] for i in range(ctx)], axis=1)
        acc = jnp.dot(xs, ws[li][...], preferred_element_type=jnp.float32)
        p = ps[li][...]
        y = jnp.maximum(acc + p[0:1, :], 0.0) * p[1:2, :] - p[2:3, :]
        x = y.astype(jnp.bfloat16)

    t_f = y.shape[0]
    m = jnp.sum(y, axis=0, keepdims=True) * (1.0 / t_f)
    d = y - m
    v = jnp.sum(d * d, axis=0, keepdims=True) * (1.0 / (t_f - 1))
    mean_ref[0] = m
    std_ref[0] = jnp.sqrt(v)


def _head_body(mean_ref, std_ref, w11a_ref, w11b_ref, b11_ref,
               wlda_ref, blda_ref, wplda_ref, bplda_ref, pq_ref, o_ref,
               *, n_pairs):
    m = mean_ref[...].astype(jnp.bfloat16)
    s = std_ref[...].astype(jnp.bfloat16)
    xv = (jnp.dot(m, w11a_ref[...], preferred_element_type=jnp.float32)
          + jnp.dot(s, w11b_ref[...], preferred_element_type=jnp.float32)
          + b11_ref[...])
    y = (jnp.dot(xv.astype(jnp.bfloat16), wlda_ref[...],
                 preferred_element_type=jnp.float32) + blda_ref[...])
    ss = jnp.sum(y * y, axis=1, keepdims=True)
    y = y * lax.rsqrt(jnp.maximum(ss, 1e-24))
    z = (jnp.dot(y.astype(jnp.bfloat16), wplda_ref[...],
                 preferred_element_type=jnp.float32) + bplda_ref[...])
    z1 = z[:n_pairs, :]
    z2 = z[n_pairs:2 * n_pairs, :]
    p = pq_ref[0:1, :]
    q = pq_ref[1:2, :]
    o_ref[...] = jnp.sum(z1 * z1 * q + z2 * z2 * q + 2.0 * z1 * z2 * p,
                         axis=1, keepdims=True)


def kernel(x1, x2, w_t_0, b_0, bn_mean_0, bn_var_0, w_t_1, b_1, bn_mean_1,
           bn_var_1, w_t_2, b_2, bn_mean_2, bn_var_2, w_t_3, b_3, bn_mean_3,
           bn_var_3, w_t_4, b_4, bn_mean_4, bn_var_4, w_t_5, b_5, bn_mean_5,
           bn_var_5, w_t_6, b_6, bn_mean_6, bn_var_6, w_t_7, b_7, bn_mean_7,
           bn_var_7, w_t_8, b_8, bn_mean_8, bn_var_8, w_t_9, b_9, bn_mean_9,
           bn_var_9, w11_t, b11, wlda_t, blda, wplda_t, bplda, p_sqrt, q):
    n_pairs = x1.shape[0]
    n_utt = 2 * n_pairs
    t_in = x1.shape[2]

    x = jnp.concatenate([x1, x2], axis=0)
    x = jnp.transpose(x, (0, 2, 1)).astype(jnp.float32)

    w_ts = [w_t_0, w_t_1, w_t_2, w_t_3, w_t_4, w_t_5, w_t_6, w_t_7, w_t_8,
            w_t_9]
    bs = [b_0, b_1, b_2, b_3, b_4, b_5, b_6, b_7, b_8, b_9]
    bn_means = [bn_mean_0, bn_mean_1, bn_mean_2, bn_mean_3, bn_mean_4,
                bn_mean_5, bn_mean_6, bn_mean_7, bn_mean_8, bn_mean_9]
    bn_vars = [bn_var_0, bn_var_1, bn_var_2, bn_var_3, bn_var_4, bn_var_5,
               bn_var_6, bn_var_7, bn_var_8, bn_var_9]

    ws = [w.astype(jnp.bfloat16) for w in w_ts]
    params = []
    for b, mu, var in zip(bs, bn_means, bn_vars):
        s = lax.rsqrt(var + _BN_EPS)
        params.append(jnp.stack([b, s, mu * s], axis=0))

    t_f = t_in
    for (_, _, ctx, dil) in _CFG:
        t_f -= dil * (ctx - 1)

    flops = sum(2 * n_utt * (t_in - dil * (ctx - 1)) * ctx * d_in * d_out
                for (d_in, d_out, ctx, dil) in _CFG)
    wbytes = sum(2 * ctx * d_in * d_out for (d_in, d_out, ctx, dil) in _CFG)

    mean, std = pl.pallas_call(
        _trunk_body,
        out_shape=[jax.ShapeDtypeStruct((n_utt, 1, 1500), jnp.float32),
                   jax.ShapeDtypeStruct((n_utt, 1, 1500), jnp.float32)],
        grid=(n_utt,),
        in_specs=(
            [pl.BlockSpec((1, t_in, 30), lambda i: (i, 0, 0))]
            + [pl.BlockSpec(w.shape, lambda i: (0, 0)) for w in ws]
            + [pl.BlockSpec(p.shape, lambda i: (0, 0)) for p in params]
        ),
        out_specs=[pl.BlockSpec((1, 1, 1500), lambda i: (i, 0, 0)),
                   pl.BlockSpec((1, 1, 1500), lambda i: (i, 0, 0))],
        compiler_params=pltpu.CompilerParams(
            dimension_semantics=("parallel",),
            vmem_limit_bytes=64 * 1024 * 1024),
        cost_estimate=pl.CostEstimate(
            flops=flops, transcendentals=0,
            bytes_accessed=wbytes + 4 * n_utt * (t_in * 30 + 2 * 1500)),
    )(x, *ws, *params)
    mean = mean.reshape(n_utt, 1500)
    std = std.reshape(n_utt, 1500)

    w11_bf = w11_t.astype(jnp.bfloat16)
    pq = jnp.stack([p_sqrt * p_sqrt, q], axis=0)

    out = pl.pallas_call(
        functools.partial(_head_body, n_pairs=n_pairs),
        out_shape=jax.ShapeDtypeStruct((n_pairs, 1), jnp.float32),
        in_specs=[pl.BlockSpec(memory_space=pltpu.MemorySpace.VMEM)] * 10,
        out_specs=pl.BlockSpec(memory_space=pltpu.MemorySpace.VMEM),
        compiler_params=pltpu.CompilerParams(
            vmem_limit_bytes=32 * 1024 * 1024),
    )(mean, std, w11_bf[:1500, :], w11_bf[1500:, :], b11.reshape(1, -1),
      wlda_t.astype(jnp.bfloat16), blda.reshape(1, -1),
      wplda_t.astype(jnp.bfloat16), bplda.reshape(1, -1), pq)
    return out.reshape(n_pairs)


# speedup vs baseline: 1.7519x; 1.7274x over previous
import functools

import jax
import jax.numpy as jnp
from jax import lax
from jax.experimental import pallas as pl
from jax.experimental.pallas import tpu as pltpu

_BN_EPS = 1e-5

_CFG = [
    (30, 512, 5, 1), (512, 512, 1, 1), (512, 512, 3, 2), (512, 512, 1, 1),
    (512, 512, 3, 3), (512, 512, 1, 1), (512, 512, 3, 4), (512, 512, 1, 1),
    (512, 512, 1, 1), (512, 1500, 1, 1),
]
_NL = len(_CFG)


def _trunk_body(x_ref, *args, n_half, t_valid, t_work, t_pad):
    ws = args[:_NL]
    bs = args[_NL:2 * _NL]
    mus = args[2 * _NL:3 * _NL]
    vas = args[3 * _NL:4 * _NL]
    mean_ref, std_ref = args[4 * _NL], args[4 * _NL + 1]

    x = x_ref[0].astype(jnp.bfloat16)
    y = None
    for li, (d_in, d_out, ctx, dil) in enumerate(_CFG):
        if ctx == 1:
            xs = x[:, 0:t_work, :]
        else:
            xs = jnp.concatenate(
                [x[:, i * dil:i * dil + t_work, :] for i in range(ctx)],
                axis=2)
        xs2 = xs.reshape(n_half * t_work, ctx * d_in)
        wb = ws[li][...].astype(jnp.bfloat16)
        acc = jnp.dot(xs2, wb, preferred_element_type=jnp.float32)
        y = (jnp.maximum(acc + bs[li][...], 0.0) - mus[li][...]) \
            * lax.rsqrt(vas[li][...] + _BN_EPS)
        if li < _NL - 1:
            yb = y.astype(jnp.bfloat16).reshape(n_half, t_work, d_out)
            x = jnp.concatenate(
                [yb, jnp.zeros((n_half, t_pad - t_work, d_out),
                               jnp.bfloat16)], axis=1)

    h = y.reshape(n_half, t_work, 1500)[:, 0:t_valid, :]
    m = jnp.sum(h, axis=1) * (1.0 / t_valid)
    d = h - m[:, None, :]
    v = jnp.sum(d * d, axis=1) * (1.0 / (t_valid - 1))
    mean_ref[0] = m
    std_ref[0] = jnp.sqrt(v)


def _head_body(mean_ref, std_ref, w11_ref, b11_ref, wlda_ref, blda_ref,
               wplda_ref, bplda_ref, psqrt_ref, q_ref, o_ref, *, n_pairs):
    m = mean_ref[...].astype(jnp.bfloat16)
    s = std_ref[...].astype(jnp.bfloat16)
    w11 = w11_ref[...].astype(jnp.bfloat16)
    xv = (jnp.dot(m, w11[0:1500, :], preferred_element_type=jnp.float32)
          + jnp.dot(s, w11[1500:3000, :], preferred_element_type=jnp.float32)
          + b11_ref[...])
    y = (jnp.dot(xv.astype(jnp.bfloat16),
                 wlda_ref[...].astype(jnp.bfloat16),
                 preferred_element_type=jnp.float32) + blda_ref[...])
    ss = jnp.sum(y * y, axis=1, keepdims=True)
    y = y * lax.rsqrt(jnp.maximum(ss, 1e-24))
    z = (jnp.dot(y.astype(jnp.bfloat16),
                 wplda_ref[...].astype(jnp.bfloat16),
                 preferred_element_type=jnp.float32) + bplda_ref[...])
    z1 = z[:n_pairs, :]
    z2 = z[n_pairs:2 * n_pairs, :]
    p = psqrt_ref[...] * psqrt_ref[...]
    q = q_ref[...]
    o_ref[...] = jnp.sum(z1 * z1 * q + z2 * z2 * q + 2.0 * z1 * z2 * p,
                         axis=1, keepdims=True)


def kernel(x1, x2, w_t_0, b_0, bn_mean_0, bn_var_0, w_t_1, b_1, bn_mean_1,
           bn_var_1, w_t_2, b_2, bn_mean_2, bn_var_2, w_t_3, b_3, bn_mean_3,
           bn_var_3, w_t_4, b_4, bn_mean_4, bn_var_4, w_t_5, b_5, bn_mean_5,
           bn_var_5, w_t_6, b_6, bn_mean_6, bn_var_6, w_t_7, b_7, bn_mean_7,
           bn_var_7, w_t_8, b_8, bn_mean_8, bn_var_8, w_t_9, b_9, bn_mean_9,
           bn_var_9, w11_t, b11, wlda_t, blda, wplda_t, bplda, p_sqrt, q):
    n_pairs = x1.shape[0]
    n_utt = 2 * n_pairs
    n_half = n_utt // 2
    t_in = x1.shape[2]

    t_valid = t_in
    for (_, _, ctx, dil) in _CFG:
        t_valid -= dil * (ctx - 1)

    x = jnp.concatenate([x1, x2], axis=0)
    x = jnp.transpose(x, (0, 2, 1)).astype(jnp.float32)
    t_pad = t_in + 8
    x = jnp.pad(x, ((0, 0), (0, t_pad - t_in), (0, 0)))
    x = x.reshape(2, n_half, t_pad, 30)

    ws = [w_t_0, w_t_1, w_t_2, w_t_3, w_t_4, w_t_5, w_t_6, w_t_7, w_t_8,
          w_t_9]
    bs = [v.reshape(1, -1) for v in
          (b_0, b_1, b_2, b_3, b_4, b_5, b_6, b_7, b_8, b_9)]
    mus = [v.reshape(1, -1) for v in
           (bn_mean_0, bn_mean_1, bn_mean_2, bn_mean_3, bn_mean_4, bn_mean_5,
            bn_mean_6, bn_mean_7, bn_mean_8, bn_mean_9)]
    vas = [v.reshape(1, -1) for v in
           (bn_var_0, bn_var_1, bn_var_2, bn_var_3, bn_var_4, bn_var_5,
            bn_var_6, bn_var_7, bn_var_8, bn_var_9)]

    flops = sum(2 * n_utt * t_in * ctx * d_in * d_out
                for (d_in, d_out, ctx, dil) in _CFG)
    wbytes = sum(4 * ctx * d_in * d_out for (d_in, d_out, ctx, dil) in _CFG)

    mean, std = pl.pallas_call(
        functools.partial(_trunk_body, n_half=n_half, t_valid=t_valid,
                          t_work=t_in, t_pad=t_pad),
        out_shape=[jax.ShapeDtypeStruct((2, n_half, 1500), jnp.float32),
                   jax.ShapeDtypeStruct((2, n_half, 1500), jnp.float32)],
        grid=(2,),
        in_specs=(
            [pl.BlockSpec((1, n_half, t_pad, 30), lambda i: (i, 0, 0, 0))]
            + [pl.BlockSpec(w.shape, lambda i: (0, 0)) for w in ws]
            + [pl.BlockSpec(v.shape, lambda i: (0, 0)) for v in bs]
            + [pl.BlockSpec(v.shape, lambda i: (0, 0)) for v in mus]
            + [pl.BlockSpec(v.shape, lambda i: (0, 0)) for v in vas]
        ),
        out_specs=[pl.BlockSpec((1, n_half, 1500), lambda i: (i, 0, 0)),
                   pl.BlockSpec((1, n_half, 1500), lambda i: (i, 0, 0))],
        compiler_params=pltpu.CompilerParams(
            dimension_semantics=("parallel",),
            vmem_limit_bytes=100 * 1024 * 1024),
        cost_estimate=pl.CostEstimate(
            flops=flops, transcendentals=0,
            bytes_accessed=wbytes + 4 * n_utt * (t_pad * 30 + 2 * 1500)),
    )(x, *ws, *bs, *mus, *vas)

    out = pl.pallas_call(
        functools.partial(_head_body, n_pairs=n_pairs),
        out_shape=jax.ShapeDtypeStruct((n_pairs, 1), jnp.float32),
        in_specs=[pl.BlockSpec(memory_space=pltpu.MemorySpace.VMEM)] * 10,
        out_specs=pl.BlockSpec(memory_space=pltpu.MemorySpace.VMEM),
        compiler_params=pltpu.CompilerParams(
            vmem_limit_bytes=64 * 1024 * 1024),
    )(mean.reshape(n_utt, 1500), std.reshape(n_utt, 1500),
      w11_t, b11.reshape(1, -1), wlda_t, blda.reshape(1, -1),
      wplda_t, bplda.reshape(1, -1), p_sqrt.reshape(1, -1), q.reshape(1, -1))
    return out.reshape(n_pairs)


# speedup vs baseline: 1.8921x; 1.0800x over previous
import functools

import jax
import jax.numpy as jnp
from jax import lax
from jax.experimental import pallas as pl
from jax.experimental.pallas import tpu as pltpu

_BN_EPS = 1e-5

_CFG = [
    (30, 512, 5, 1), (512, 512, 1, 1), (512, 512, 3, 2), (512, 512, 1, 1),
    (512, 512, 3, 3), (512, 512, 1, 1), (512, 512, 3, 4), (512, 512, 1, 1),
    (512, 512, 1, 1), (512, 1500, 1, 1),
]
_NL = len(_CFG)
_NEEDS_PAD_OUT = {li for li in range(_NL - 1) if _CFG[li + 1][2] > 1}


def _trunk_body(x_ref, *args, n_half, t_valid, t_work, t_pad):
    ws = args[:_NL]
    bs = args[_NL:2 * _NL]
    mus = args[2 * _NL:3 * _NL]
    vas = args[3 * _NL:4 * _NL]
    mean_ref, std_ref = args[4 * _NL], args[4 * _NL + 1]

    n_c = n_half // 2
    pad_rows = t_pad - t_work

    chains = [x_ref[0, 0:n_c].astype(jnp.bfloat16),
              x_ref[0, n_c:n_half].astype(jnp.bfloat16)]
    finals = [None, None]

    for li, (d_in, d_out, ctx, dil) in enumerate(_CFG):
        wb = ws[li][...].astype(jnp.bfloat16)
        b = bs[li][...]
        mu = mus[li][...]
        inv_sig = lax.rsqrt(vas[li][...] + _BN_EPS)
        for ci in range(2):
            x = chains[ci]
            if ctx == 1:
                xs = x[:, 0:t_work, :]
            else:
                xs = jnp.concatenate(
                    [x[:, i * dil:i * dil + t_work, :] for i in range(ctx)],
                    axis=2)
            xs2 = xs.reshape(n_c * t_work, ctx * d_in)
            acc = jnp.dot(xs2, wb, preferred_element_type=jnp.float32)
            if li == _NL - 1:
                finals[ci] = jnp.maximum(acc + b, 0.0)
                continue
            y = (jnp.maximum(acc + b, 0.0) - mu) * inv_sig
            yb = y.astype(jnp.bfloat16).reshape(n_c, t_work, d_out)
            if li in _NEEDS_PAD_OUT:
                yb = jnp.concatenate(
                    [yb, jnp.zeros((n_c, pad_rows, d_out), jnp.bfloat16)],
                    axis=1)
            chains[ci] = yb

    mu9 = mus[_NL - 1][...]
    s9 = lax.rsqrt(vas[_NL - 1][...] + _BN_EPS)
    for ci in range(2):
        r = finals[ci].reshape(n_c, t_work, 1500)[:, 0:t_valid, :]
        sm = jnp.sum(r, axis=1) * (1.0 / t_valid)
        sq = jnp.sum(r * r, axis=1)
        var = (sq - (float(t_valid) * sm) * sm) * (1.0 / (t_valid - 1))
        var = jnp.maximum(var, 0.0)
        row0 = ci * n_c
        mean_ref[0, row0:row0 + n_c] = (sm - mu9) * s9
        std_ref[0, row0:row0 + n_c] = jnp.sqrt(var) * s9


def _head_body(mean_ref, std_ref, w11_ref, b11_ref, wlda_ref, blda_ref,
               wplda_ref, bplda_ref, psqrt_ref, q_ref, o_ref, *, n_pairs):
    m = mean_ref[...].astype(jnp.bfloat16)
    s = std_ref[...].astype(jnp.bfloat16)
    w11 = w11_ref[...].astype(jnp.bfloat16)
    xv = (jnp.dot(m, w11[0:1500, :], preferred_element_type=jnp.float32)
          + jnp.dot(s, w11[1500:3000, :], preferred_element_type=jnp.float32)
          + b11_ref[...])
    y = (jnp.dot(xv.astype(jnp.bfloat16),
                 wlda_ref[...].astype(jnp.bfloat16),
                 preferred_element_type=jnp.float32) + blda_ref[...])
    ss = jnp.sum(y * y, axis=1, keepdims=True)
    y = y * lax.rsqrt(jnp.maximum(ss, 1e-24))
    z = (jnp.dot(y.astype(jnp.bfloat16),
                 wplda_ref[...].astype(jnp.bfloat16),
                 preferred_element_type=jnp.float32) + bplda_ref[...])
    z1 = z[:n_pairs, :]
    z2 = z[n_pairs:2 * n_pairs, :]
    p = psqrt_ref[...] * psqrt_ref[...]
    q = q_ref[...]
    o_ref[...] = jnp.sum(z1 * z1 * q + z2 * z2 * q + 2.0 * z1 * z2 * p,
                         axis=1, keepdims=True)


def kernel(x1, x2, w_t_0, b_0, bn_mean_0, bn_var_0, w_t_1, b_1, bn_mean_1,
           bn_var_1, w_t_2, b_2, bn_mean_2, bn_var_2, w_t_3, b_3, bn_mean_3,
           bn_var_3, w_t_4, b_4, bn_mean_4, bn_var_4, w_t_5, b_5, bn_mean_5,
           bn_var_5, w_t_6, b_6, bn_mean_6, bn_var_6, w_t_7, b_7, bn_mean_7,
           bn_var_7, w_t_8, b_8, bn_mean_8, bn_var_8, w_t_9, b_9, bn_mean_9,
           bn_var_9, w11_t, b11, wlda_t, blda, wplda_t, bplda, p_sqrt, q):
    n_pairs = x1.shape[0]
    n_utt = 2 * n_pairs
    n_half = n_utt // 2
    t_in = x1.shape[2]

    t_valid = t_in
    for (_, _, ctx, dil) in _CFG:
        t_valid -= dil * (ctx - 1)

    x = jnp.concatenate([x1, x2], axis=0)
    x = jnp.transpose(x, (0, 2, 1)).astype(jnp.float32)
    t_pad = t_in + 8
    x = jnp.pad(x, ((0, 0), (0, t_pad - t_in), (0, 0)))
    x = x.reshape(2, n_half, t_pad, 30)

    ws = [w_t_0, w_t_1, w_t_2, w_t_3, w_t_4, w_t_5, w_t_6, w_t_7, w_t_8,
          w_t_9]
    bs = [v.reshape(1, -1) for v in
          (b_0, b_1, b_2, b_3, b_4, b_5, b_6, b_7, b_8, b_9)]
    mus = [v.reshape(1, -1) for v in
           (bn_mean_0, bn_mean_1, bn_mean_2, bn_mean_3, bn_mean_4, bn_mean_5,
            bn_mean_6, bn_mean_7, bn_mean_8, bn_mean_9)]
    vas = [v.reshape(1, -1) for v in
           (bn_var_0, bn_var_1, bn_var_2, bn_var_3, bn_var_4, bn_var_5,
            bn_var_6, bn_var_7, bn_var_8, bn_var_9)]

    flops = sum(2 * n_utt * t_in * ctx * d_in * d_out
                for (d_in, d_out, ctx, dil) in _CFG)
    wbytes = sum(4 * ctx * d_in * d_out for (d_in, d_out, ctx, dil) in _CFG)

    mean, std = pl.pallas_call(
        functools.partial(_trunk_body, n_half=n_half, t_valid=t_valid,
                          t_work=t_in, t_pad=t_pad),
        out_shape=[jax.ShapeDtypeStruct((2, n_half, 1500), jnp.float32),
                   jax.ShapeDtypeStruct((2, n_half, 1500), jnp.float32)],
        grid=(2,),
        in_specs=(
            [pl.BlockSpec((1, n_half, t_pad, 30), lambda i: (i, 0, 0, 0))]
            + [pl.BlockSpec(w.shape, lambda i: (0, 0)) for w in ws]
            + [pl.BlockSpec(v.shape, lambda i: (0, 0)) for v in bs]
            + [pl.BlockSpec(v.shape, lambda i: (0, 0)) for v in mus]
            + [pl.BlockSpec(v.shape, lambda i: (0, 0)) for v in vas]
        ),
        out_specs=[pl.BlockSpec((1, n_half, 1500), lambda i: (i, 0, 0)),
                   pl.BlockSpec((1, n_half, 1500), lambda i: (i, 0, 0))],
        compiler_params=pltpu.CompilerParams(
            dimension_semantics=("parallel",),
            vmem_limit_bytes=100 * 1024 * 1024),
        cost_estimate=pl.CostEstimate(
            flops=flops, transcendentals=0,
            bytes_accessed=wbytes + 4 * n_utt * (t_pad * 30 + 2 * 1500)),
    )(x, *ws, *bs, *mus, *vas)

    out = pl.pallas_call(
        functools.partial(_head_body, n_pairs=n_pairs),
        out_shape=jax.ShapeDtypeStruct((n_pairs, 1), jnp.float32),
        in_specs=[pl.BlockSpec(memory_space=pltpu.MemorySpace.VMEM)] * 10,
        out_specs=pl.BlockSpec(memory_space=pltpu.MemorySpace.VMEM),
        compiler_params=pltpu.CompilerParams(
            vmem_limit_bytes=64 * 1024 * 1024),
    )(mean.reshape(n_utt, 1500), std.reshape(n_utt, 1500),
      w11_t, b11.reshape(1, -1), wlda_t, blda.reshape(1, -1),
      wplda_t, bplda.reshape(1, -1), p_sqrt.reshape(1, -1), q.reshape(1, -1))
    return out.reshape(n_pairs)
